# baseline (speedup 1.0000x reference)
import functools

import jax
import jax.numpy as jnp
from jax import lax
from jax.experimental import pallas as pl
from jax.experimental.pallas import tpu as pltpu
from jax.experimental.pallas import tpu_sc as plsc

N = 10000
K = 16
D = 256
NC = 2
NS = 16
NW = NC * NS
NPAD = 10240
EPAD = NPAD * K

GM_NT = (640, 0)
GM_CN = 40
GM_CE = GM_CN * K
GM_BASE = (0, NS * GM_NT[0])
GE_NT = (448, 192)
GE_CN = 32
GE_CE = GE_CN * K
GE_BASE = (0, NS * GE_NT[0])

assert NS * (GM_NT[0] + GM_NT[1]) == NPAD
assert NS * (GE_NT[0] + GE_NT[1]) == NPAD

_mesh = plsc.VectorSubcoreMesh(
    core_axis_name="c", subcore_axis_name="s", num_cores=NC, num_subcores=NS)


@functools.partial(
    pl.kernel,
    out_type=jax.ShapeDtypeStruct((EPAD * 4,), jnp.float32),
    mesh=_mesh,
    compiler_params=pltpu.CompilerParams(
        use_tc_tiling_on_sc=False, needs_layout_passes=False),
    scratch_types=[
        [pltpu.VMEM((GE_CE,), jnp.int32)] * 2
        + [pltpu.VMEM((GE_CE,), jnp.float32)] * 12
        + [pltpu.VMEM((GE_CE * 4,), jnp.float32)],
        pltpu.SemaphoreType.DMA,
        pltpu.SemaphoreType.DMA,
    ],
)
def _sc_geo(xh, yh, zh, cxh, cyh, czh, kih, geoh, bufs, sem0, sem1):
    cid = lax.axis_index("c")
    sid = lax.axis_index("s")
    idx = bufs[0:2]
    nx, ny, nz = bufs[2:4], bufs[4:6], bufs[6:8]
    cx, cy, cz = bufs[8:10], bufs[10:12], bufs[12:14]
    gv = bufs[14]
    sems = (sem0, sem1)

    def fire(ch, b, tile_node0):
        base = (tile_node0 + ch * GE_CN) * K
        pltpu.sync_copy(kih.at[pl.ds(base, GE_CE)], idx[b])
        pltpu.async_copy(xh.at[idx[b]], nx[b], sems[b])
        pltpu.async_copy(yh.at[idx[b]], ny[b], sems[b])
        pltpu.async_copy(zh.at[idx[b]], nz[b], sems[b])
        pltpu.sync_copy(cxh.at[pl.ds(base, GE_CE)], cx[b])
        pltpu.sync_copy(cyh.at[pl.ds(base, GE_CE)], cy[b])
        pltpu.sync_copy(czh.at[pl.ds(base, GE_CE)], cz[b])

    for core in range(NC):
        nchunk = GE_NT[core] // GE_CN
        npair = nchunk // 2
        tile_node0 = GE_BASE[core] + sid * GE_NT[core]

        @pl.when(cid == core)
        def _(tile_node0=tile_node0, nchunk=nchunk, npair=npair):
            fire(0, 0, tile_node0)
            fire(1, 1, tile_node0)

            def pair(ch2, carry):
                for b in (0, 1):
                    ch = 2 * ch2 + b
                    pltpu.make_async_copy(xh.at[idx[b]], nx[b], sems[b]).wait()
                    pltpu.make_async_copy(yh.at[idx[b]], ny[b], sems[b]).wait()
                    pltpu.make_async_copy(zh.at[idx[b]], nz[b], sems[b]).wait()
                    nxb, nyb, nzb = nx[b], ny[b], nz[b]
                    cxb, cyb, czb = cx[b], cy[b], cz[b]

                    def comp(v, c2):
                        s = pl.ds(v * 16, 16)
                        dx = cxb[s] - nxb[s]
                        dy = cyb[s] - nyb[s]
                        dz = czb[s] - nzb[s]
                        sq = dx * dx + dy * dy + dz * dz
                        idx4 = v * 64 + lax.iota(jnp.int32, 16) * 4
                        plsc.store_scatter(gv, [idx4], dx)
                        plsc.store_scatter(gv, [idx4 + 1], dy)
                        plsc.store_scatter(gv, [idx4 + 2], dz)
                        plsc.store_scatter(gv, [idx4 + 3], sq)
                        return c2

                    lax.fori_loop(0, GE_CE // 16, comp, 0)
                    base = (tile_node0 + ch * GE_CN) * K
                    pltpu.sync_copy(gv, geoh.at[pl.ds(base * 4, GE_CE * 4)])

                    @pl.when(ch2 < npair - 1)
                    def _():
                        fire(ch + 2, b, tile_node0)
                return carry

            lax.fori_loop(0, npair, pair, 0)


@functools.partial(
    pl.kernel,
    out_type=jax.ShapeDtypeStruct((NPAD * 64,), jnp.float32),
    mesh=_mesh,
    compiler_params=pltpu.CompilerParams(
        use_tc_tiling_on_sc=False, needs_layout_passes=False),
    scratch_types=[
        pltpu.VMEM((GM_CE,), jnp.int32),
        pltpu.VMEM((GM_CE,), jnp.int32),
        pltpu.VMEM((GM_CE, 64), jnp.float32),
        pltpu.VMEM((GM_CE, 64), jnp.float32),
        pltpu.VMEM((GM_CN * 64,), jnp.float32),
        pltpu.SemaphoreType.DMA,
        pltpu.SemaphoreType.DMA,
    ],
)
def _sc_gmean(th, kih, oh, idx0, idx1, rows0, rows1, acc, sem0, sem1):
    cid = lax.axis_index("c")
    sid = lax.axis_index("s")
    idx = (idx0, idx1)
    rows = (rows0, rows1)
    sems = (sem0, sem1)
    nchunk = GM_NT[0] // GM_CN
    tile_node0 = sid * GM_NT[0]

    def fire(ch, b):
        ebase = (tile_node0 + ch * GM_CN) * K
        pltpu.sync_copy(kih.at[pl.ds(ebase, GM_CE)], idx[b])
        return pltpu.async_copy(th.at[idx[b]], rows[b], sems[b])

    @pl.when(cid == 0)
    def _():
        handles = [fire(0, 0), None]
        for ch in range(nchunk):
            cur = ch % 2
            if ch + 1 < nchunk:
                handles[1 - cur] = fire(ch + 1, 1 - cur)
            handles[cur].wait()
            rowsb = rows[cur]

            def comp(n, c2):
                for c in range(4):
                    sl = pl.ds(c * 16, 16)
                    r = [rowsb[n * 16 + j, sl] for j in range(16)]
                    while len(r) > 1:
                        r = [r[2 * i] + r[2 * i + 1]
                             for i in range(len(r) // 2)]
                    acc[pl.ds(n * 64 + c * 16, 16)] = r[0] * (1.0 / 16.0)
                return c2

            lax.fori_loop(0, GM_CN, comp, 0, unroll=2)
            nbase = tile_node0 + ch * GM_CN
            pltpu.sync_copy(acc, oh.at[pl.ds(nbase * 64, GM_CN * 64)])


def _lrelu(v):
    return jnp.where(v >= 0, v, 0.2 * v)


_BE = 2048
_BNn = _BE // K


def _tc1_body(geo, w4a, w4b, b1g, b2g, pm, g1o, g2o):
    g = geo[...]
    col = lax.broadcasted_iota(jnp.int32, (_BE, 4), 1)
    gf = jnp.where(col == 3, jnp.sqrt(jnp.maximum(g, 0.0)), g)
    z1 = jnp.dot(gf, w4a[...], preferred_element_type=jnp.float32) + b1g[...]
    z2 = jnp.dot(gf, w4b[...], preferred_element_type=jnp.float32) + b2g[...]
    z1 = _lrelu(z1)
    z2 = _lrelu(z2)
    p = pm[...]
    g1o[...] = jnp.dot(p, z1, preferred_element_type=jnp.float32)
    g2o[...] = jnp.dot(p, z2, preferred_element_type=jnp.float32)


def _tc1(geo, w4a, w4b, b1g, b2g, pm):
    return pl.pallas_call(
        _tc1_body,
        grid=(EPAD // _BE,),
        in_specs=[
            pl.BlockSpec((_BE, 4), lambda i: (i, 0)),
            pl.BlockSpec((4, 64), lambda i: (0, 0)),
            pl.BlockSpec((4, 128), lambda i: (0, 0)),
            pl.BlockSpec((1, 64), lambda i: (0, 0)),
            pl.BlockSpec((1, 128), lambda i: (0, 0)),
            pl.BlockSpec((_BNn, _BE), lambda i: (0, 0)),
        ],
        out_specs=[
            pl.BlockSpec((_BNn, 64), lambda i: (i, 0)),
            pl.BlockSpec((_BNn, 128), lambda i: (i, 0)),
        ],
        out_shape=[
            jax.ShapeDtypeStruct((NPAD, 64), jnp.float32),
            jax.ShapeDtypeStruct((NPAD, 128), jnp.float32),
        ],
    )(geo, w4a, w4b, b1g, b2g, pm)


_BN = 512


def _tc2_body(g2m, gg1, w2at, w2bt, b2r, c1r, rbr, w1t, b1r, p_o, x1_o, h2_o):
    P = (jnp.dot(g2m[...], w2at[...], preferred_element_type=jnp.float32)
         + jnp.dot(gg1[...], w2bt[...], preferred_element_type=jnp.float32)
         + b2r[...])
    x1 = _lrelu(P + c1r[...]) + rbr[...]
    h2 = _lrelu(jnp.dot(x1, w1t[...], preferred_element_type=jnp.float32)
                + b1r[...])
    p_o[...] = P
    x1_o[...] = x1
    h2_o[...] = h2


def _tc2(g2m, gg1, w2at, w2bt, b2r, c1r, rbr, w1t, b1r):
    return pl.pallas_call(
        _tc2_body,
        grid=(NPAD // _BN,),
        in_specs=[
            pl.BlockSpec((_BN, 128), lambda i: (i, 0)),
            pl.BlockSpec((_BN, 64), lambda i: (i, 0)),
            pl.BlockSpec((128, D), lambda i: (0, 0)),
            pl.BlockSpec((64, D), lambda i: (0, 0)),
            pl.BlockSpec((1, D), lambda i: (0, 0)),
            pl.BlockSpec((1, D), lambda i: (0, 0)),
            pl.BlockSpec((1, D), lambda i: (0, 0)),
            pl.BlockSpec((D, 64), lambda i: (0, 0)),
            pl.BlockSpec((1, 64), lambda i: (0, 0)),
        ],
        out_specs=[
            pl.BlockSpec((_BN, D), lambda i: (i, 0)),
            pl.BlockSpec((_BN, D), lambda i: (i, 0)),
            pl.BlockSpec((_BN, 64), lambda i: (i, 0)),
        ],
        out_shape=[
            jax.ShapeDtypeStruct((NPAD, D), jnp.float32),
            jax.ShapeDtypeStruct((NPAD, D), jnp.float32),
            jax.ShapeDtypeStruct((NPAD, 64), jnp.float32),
        ],
    )(g2m, gg1, w2at, w2bt, b2r, c1r, rbr, w1t, b1r)


def _tc3_body(p, x1, hh, w2ct, wrt, brr, x2_o):
    t = _lrelu(p[...] + jnp.dot(hh[...], w2ct[...],
                                preferred_element_type=jnp.float32))
    x2_o[...] = t + _lrelu(jnp.dot(x1[...], wrt[...],
                                   preferred_element_type=jnp.float32)
                           + brr[...])


def _tc3(p, x1, hh, w2ct, wrt, brr):
    return pl.pallas_call(
        _tc3_body,
        grid=(NPAD // _BN,),
        in_specs=[
            pl.BlockSpec((_BN, D), lambda i: (i, 0)),
            pl.BlockSpec((_BN, D), lambda i: (i, 0)),
            pl.BlockSpec((_BN, 64), lambda i: (i, 0)),
            pl.BlockSpec((64, D), lambda i: (0, 0)),
            pl.BlockSpec((D, D), lambda i: (0, 0)),
            pl.BlockSpec((1, D), lambda i: (0, 0)),
        ],
        out_specs=pl.BlockSpec((_BN, D), lambda i: (i, 0)),
        out_shape=jax.ShapeDtypeStruct((NPAD, D), jnp.float32),
    )(p, x1, hh, w2ct, wrt, brr)


def kernel(inputs, knn, W1, b1, Wg1, bg1, Wg2, bg2, W2, b2, Wr, br):
    f32 = jnp.float32
    knn_p = jnp.pad(knn.astype(jnp.int32), ((0, NPAD - N), (0, 0)))
    kflat = knn_p.reshape(-1)
    ip = jnp.pad(inputs.astype(f32), ((0, NPAD - N), (0, 0)))
    x_, y_, z_ = ip[:, 0], ip[:, 1], ip[:, 2]
    cx = jnp.repeat(x_, K)
    cy = jnp.repeat(y_, K)
    cz = jnp.repeat(z_, K)

    w4a = Wg1.T.astype(f32)
    w4b = Wg2.T.astype(f32)
    b1g = bg1.reshape(1, 64).astype(f32)
    b2g = bg2.reshape(1, 128).astype(f32)
    pm = (jnp.repeat(jnp.eye(_BNn, dtype=f32), K, axis=1) / K)

    h1 = _lrelu(b1)
    c1r = (W2[:, 192:] @ h1).reshape(1, D).astype(f32)
    rbr = _lrelu(br).reshape(1, D).astype(f32)
    b2r = b2.reshape(1, D).astype(f32)
    b1r = b1.reshape(1, 64).astype(f32)
    brr = br.reshape(1, D).astype(f32)
    w2at = W2[:, :128].T.astype(f32)
    w2bt = W2[:, 128:192].T.astype(f32)
    w2ct = W2[:, 192:].T.astype(f32)
    w1t = W1.T.astype(f32)
    wrt = Wr.T.astype(f32)

    geo = _sc_geo(x_, y_, z_, cx, cy, cz, kflat).reshape(EPAD, 4)
    g1m, g2m = _tc1(geo, w4a, w4b, b1g, b2g, pm)
    gg1 = _sc_gmean(g1m, kflat).reshape(NPAD, 64)
    P, x1, h2 = _tc2(g2m, gg1, w2at, w2bt, b2r, c1r, rbr, w1t, b1r)
    hm2 = _sc_gmean(h2, kflat).reshape(NPAD, 64)
    hh = _sc_gmean(hm2, kflat).reshape(NPAD, 64)
    x2 = _tc3(P, x1, hh, w2ct, wrt, brr)
    return x2[:N]


# speedup vs baseline: 1.7304x; 1.7304x over previous
import functools

import jax
import jax.numpy as jnp
from jax import lax
from jax.experimental import pallas as pl
from jax.experimental.pallas import tpu as pltpu
from jax.experimental.pallas import tpu_sc as plsc

N = 10000
K = 16
D = 256
NC = 2
NS = 16
NW = NC * NS
NPAD = 10240
EPAD = NPAD * K

GM_NT = (320, 320)
GM_CN = 40
GM_CE = GM_CN * K
GM_BASE = (0, NS * GM_NT[0])
GE_NT = (448, 192)
GE_CN = 32
GE_CE = GE_CN * K
GE_BASE = (0, NS * GE_NT[0])

assert NS * (GM_NT[0] + GM_NT[1]) == NPAD
assert NS * (GE_NT[0] + GE_NT[1]) == NPAD

_mesh = plsc.VectorSubcoreMesh(
    core_axis_name="c", subcore_axis_name="s", num_cores=NC, num_subcores=NS)


@functools.partial(
    pl.kernel,
    out_type=jax.ShapeDtypeStruct((EPAD * 4,), jnp.float32),
    mesh=_mesh,
    compiler_params=pltpu.CompilerParams(
        use_tc_tiling_on_sc=False, needs_layout_passes=False),
    scratch_types=[
        [pltpu.VMEM((GE_CE,), jnp.int32)] * 2
        + [pltpu.VMEM((GE_CE,), jnp.float32)] * 12
        + [pltpu.VMEM((GE_CE * 4,), jnp.float32)],
        pltpu.SemaphoreType.DMA,
        pltpu.SemaphoreType.DMA,
    ],
)
def _sc_geo(xh, yh, zh, cxh, cyh, czh, kih, geoh, bufs, sem0, sem1):
    cid = lax.axis_index("c")
    sid = lax.axis_index("s")
    idx = bufs[0:2]
    nx, ny, nz = bufs[2:4], bufs[4:6], bufs[6:8]
    cx, cy, cz = bufs[8:10], bufs[10:12], bufs[12:14]
    gv = bufs[14]
    sems = (sem0, sem1)

    def fire(ch, b, tile_node0):
        base = (tile_node0 + ch * GE_CN) * K
        pltpu.sync_copy(kih.at[pl.ds(base, GE_CE)], idx[b])
        pltpu.async_copy(xh.at[idx[b]], nx[b], sems[b])
        pltpu.async_copy(yh.at[idx[b]], ny[b], sems[b])
        pltpu.async_copy(zh.at[idx[b]], nz[b], sems[b])
        pltpu.sync_copy(cxh.at[pl.ds(base, GE_CE)], cx[b])
        pltpu.sync_copy(cyh.at[pl.ds(base, GE_CE)], cy[b])
        pltpu.sync_copy(czh.at[pl.ds(base, GE_CE)], cz[b])

    for core in range(NC):
        nchunk = GE_NT[core] // GE_CN
        npair = nchunk // 2
        tile_node0 = GE_BASE[core] + sid * GE_NT[core]

        @pl.when(cid == core)
        def _(tile_node0=tile_node0, nchunk=nchunk, npair=npair):
            fire(0, 0, tile_node0)
            fire(1, 1, tile_node0)

            def pair(ch2, carry):
                for b in (0, 1):
                    ch = 2 * ch2 + b
                    pltpu.make_async_copy(xh.at[idx[b]], nx[b], sems[b]).wait()
                    pltpu.make_async_copy(yh.at[idx[b]], ny[b], sems[b]).wait()
                    pltpu.make_async_copy(zh.at[idx[b]], nz[b], sems[b]).wait()
                    nxb, nyb, nzb = nx[b], ny[b], nz[b]
                    cxb, cyb, czb = cx[b], cy[b], cz[b]

                    def comp(v, c2):
                        s = pl.ds(v * 16, 16)
                        dx = cxb[s] - nxb[s]
                        dy = cyb[s] - nyb[s]
                        dz = czb[s] - nzb[s]
                        sq = dx * dx + dy * dy + dz * dz
                        idx4 = v * 64 + lax.iota(jnp.int32, 16) * 4
                        plsc.store_scatter(gv, [idx4], dx)
                        plsc.store_scatter(gv, [idx4 + 1], dy)
                        plsc.store_scatter(gv, [idx4 + 2], dz)
                        plsc.store_scatter(gv, [idx4 + 3], sq)
                        return c2

                    lax.fori_loop(0, GE_CE // 16, comp, 0)
                    base = (tile_node0 + ch * GE_CN) * K
                    pltpu.sync_copy(gv, geoh.at[pl.ds(base * 4, GE_CE * 4)])

                    @pl.when(ch2 < npair - 1)
                    def _():
                        fire(ch + 2, b, tile_node0)
                return carry

            lax.fori_loop(0, npair, pair, 0)


@functools.partial(
    pl.kernel,
    out_type=jax.ShapeDtypeStruct((NPAD * 64,), jnp.float32),
    mesh=_mesh,
    compiler_params=pltpu.CompilerParams(
        use_tc_tiling_on_sc=False, needs_layout_passes=False),
    scratch_types=[
        pltpu.VMEM_SHARED((NPAD, 64), jnp.float32),
        pltpu.VMEM((GM_CE,), jnp.int32),
        pltpu.VMEM((GM_CE,), jnp.int32),
        pltpu.VMEM((GM_CE, 64), jnp.float32),
        pltpu.VMEM((GM_CE, 64), jnp.float32),
        pltpu.VMEM((GM_CN * 64,), jnp.float32),
        pltpu.SemaphoreType.DMA,
        pltpu.SemaphoreType.DMA,
    ],
)
def _sc_gmean(th, kih, oh, tab_s, idx0, idx1, rows0, rows1, acc, sem0, sem1):
    cid = lax.axis_index("c")
    sid = lax.axis_index("s")
    idx = (idx0, idx1)
    rows = (rows0, rows1)
    sems = (sem0, sem1)
    nchunk = GM_NT[0] // GM_CN
    wid = sid * NC + cid
    tile_node0 = wid * GM_NT[0]

    @pl.when(sid == 0)
    def _():
        pltpu.sync_copy(th, tab_s)

    plsc.subcore_barrier()

    def fire(ch, b):
        ebase = (tile_node0 + ch * GM_CN) * K
        pltpu.sync_copy(kih.at[pl.ds(ebase, GM_CE)], idx[b])
        return pltpu.async_copy(tab_s.at[idx[b]], rows[b], sems[b])

    handles = [fire(0, 0), None]
    for ch in range(nchunk):
        cur = ch % 2
        if ch + 1 < nchunk:
            handles[1 - cur] = fire(ch + 1, 1 - cur)
        handles[cur].wait()
        rowsb = rows[cur]

        def comp(n, c2):
            for c in range(4):
                sl = pl.ds(c * 16, 16)
                r = [rowsb[n * 16 + j, sl] for j in range(16)]
                while len(r) > 1:
                    r = [r[2 * i] + r[2 * i + 1]
                         for i in range(len(r) // 2)]
                acc[pl.ds(n * 64 + c * 16, 16)] = r[0] * (1.0 / 16.0)
            return c2

        lax.fori_loop(0, GM_CN, comp, 0, unroll=2)
        nbase = tile_node0 + ch * GM_CN
        pltpu.sync_copy(acc, oh.at[pl.ds(nbase * 64, GM_CN * 64)])


def _lrelu(v):
    return jnp.where(v >= 0, v, 0.2 * v)


_BE = 2048
_BNn = _BE // K


def _tc1_body(geo, w4a, w4b, b1g, b2g, pm, g1o, g2o):
    g = geo[...]
    col = lax.broadcasted_iota(jnp.int32, (_BE, 4), 1)
    gf = jnp.where(col == 3, jnp.sqrt(jnp.maximum(g, 0.0)), g)
    z1 = jnp.dot(gf, w4a[...], preferred_element_type=jnp.float32) + b1g[...]
    z2 = jnp.dot(gf, w4b[...], preferred_element_type=jnp.float32) + b2g[...]
    z1 = _lrelu(z1)
    z2 = _lrelu(z2)
    p = pm[...]
    g1o[...] = jnp.dot(p, z1, preferred_element_type=jnp.float32)
    g2o[...] = jnp.dot(p, z2, preferred_element_type=jnp.float32)


def _tc1(geo, w4a, w4b, b1g, b2g, pm):
    return pl.pallas_call(
        _tc1_body,
        grid=(EPAD // _BE,),
        in_specs=[
            pl.BlockSpec((_BE, 4), lambda i: (i, 0)),
            pl.BlockSpec((4, 64), lambda i: (0, 0)),
            pl.BlockSpec((4, 128), lambda i: (0, 0)),
            pl.BlockSpec((1, 64), lambda i: (0, 0)),
            pl.BlockSpec((1, 128), lambda i: (0, 0)),
            pl.BlockSpec((_BNn, _BE), lambda i: (0, 0)),
        ],
        out_specs=[
            pl.BlockSpec((_BNn, 64), lambda i: (i, 0)),
            pl.BlockSpec((_BNn, 128), lambda i: (i, 0)),
        ],
        out_shape=[
            jax.ShapeDtypeStruct((NPAD, 64), jnp.float32),
            jax.ShapeDtypeStruct((NPAD, 128), jnp.float32),
        ],
    )(geo, w4a, w4b, b1g, b2g, pm)


_BN = 512


def _tc2_body(g2m, gg1, w2at, w2bt, b2r, c1r, rbr, w1t, b1r, p_o, x1_o, h2_o):
    P = (jnp.dot(g2m[...], w2at[...], preferred_element_type=jnp.float32)
         + jnp.dot(gg1[...], w2bt[...], preferred_element_type=jnp.float32)
         + b2r[...])
    x1 = _lrelu(P + c1r[...]) + rbr[...]
    h2 = _lrelu(jnp.dot(x1, w1t[...], preferred_element_type=jnp.float32)
                + b1r[...])
    p_o[...] = P
    x1_o[...] = x1
    h2_o[...] = h2


def _tc2(g2m, gg1, w2at, w2bt, b2r, c1r, rbr, w1t, b1r):
    return pl.pallas_call(
        _tc2_body,
        grid=(NPAD // _BN,),
        in_specs=[
            pl.BlockSpec((_BN, 128), lambda i: (i, 0)),
            pl.BlockSpec((_BN, 64), lambda i: (i, 0)),
            pl.BlockSpec((128, D), lambda i: (0, 0)),
            pl.BlockSpec((64, D), lambda i: (0, 0)),
            pl.BlockSpec((1, D), lambda i: (0, 0)),
            pl.BlockSpec((1, D), lambda i: (0, 0)),
            pl.BlockSpec((1, D), lambda i: (0, 0)),
            pl.BlockSpec((D, 64), lambda i: (0, 0)),
            pl.BlockSpec((1, 64), lambda i: (0, 0)),
        ],
        out_specs=[
            pl.BlockSpec((_BN, D), lambda i: (i, 0)),
            pl.BlockSpec((_BN, D), lambda i: (i, 0)),
            pl.BlockSpec((_BN, 64), lambda i: (i, 0)),
        ],
        out_shape=[
            jax.ShapeDtypeStruct((NPAD, D), jnp.float32),
            jax.ShapeDtypeStruct((NPAD, D), jnp.float32),
            jax.ShapeDtypeStruct((NPAD, 64), jnp.float32),
        ],
    )(g2m, gg1, w2at, w2bt, b2r, c1r, rbr, w1t, b1r)


def _tc3_body(p, x1, hh, w2ct, wrt, brr, x2_o):
    t = _lrelu(p[...] + jnp.dot(hh[...], w2ct[...],
                                preferred_element_type=jnp.float32))
    x2_o[...] = t + _lrelu(jnp.dot(x1[...], wrt[...],
                                   preferred_element_type=jnp.float32)
                           + brr[...])


def _tc3(p, x1, hh, w2ct, wrt, brr):
    return pl.pallas_call(
        _tc3_body,
        grid=(NPAD // _BN,),
        in_specs=[
            pl.BlockSpec((_BN, D), lambda i: (i, 0)),
            pl.BlockSpec((_BN, D), lambda i: (i, 0)),
            pl.BlockSpec((_BN, 64), lambda i: (i, 0)),
            pl.BlockSpec((64, D), lambda i: (0, 0)),
            pl.BlockSpec((D, D), lambda i: (0, 0)),
            pl.BlockSpec((1, D), lambda i: (0, 0)),
        ],
        out_specs=pl.BlockSpec((_BN, D), lambda i: (i, 0)),
        out_shape=jax.ShapeDtypeStruct((NPAD, D), jnp.float32),
    )(p, x1, hh, w2ct, wrt, brr)


def kernel(inputs, knn, W1, b1, Wg1, bg1, Wg2, bg2, W2, b2, Wr, br):
    f32 = jnp.float32
    knn_p = jnp.pad(knn.astype(jnp.int32), ((0, NPAD - N), (0, 0)))
    kflat = knn_p.reshape(-1)
    ip = jnp.pad(inputs.astype(f32), ((0, NPAD - N), (0, 0)))
    x_, y_, z_ = ip[:, 0], ip[:, 1], ip[:, 2]
    cx = jnp.repeat(x_, K)
    cy = jnp.repeat(y_, K)
    cz = jnp.repeat(z_, K)

    w4a = Wg1.T.astype(f32)
    w4b = Wg2.T.astype(f32)
    b1g = bg1.reshape(1, 64).astype(f32)
    b2g = bg2.reshape(1, 128).astype(f32)
    pm = (jnp.repeat(jnp.eye(_BNn, dtype=f32), K, axis=1) / K)

    h1 = _lrelu(b1)
    c1r = (W2[:, 192:] @ h1).reshape(1, D).astype(f32)
    rbr = _lrelu(br).reshape(1, D).astype(f32)
    b2r = b2.reshape(1, D).astype(f32)
    b1r = b1.reshape(1, 64).astype(f32)
    brr = br.reshape(1, D).astype(f32)
    w2at = W2[:, :128].T.astype(f32)
    w2bt = W2[:, 128:192].T.astype(f32)
    w2ct = W2[:, 192:].T.astype(f32)
    w1t = W1.T.astype(f32)
    wrt = Wr.T.astype(f32)

    geo = _sc_geo(x_, y_, z_, cx, cy, cz, kflat).reshape(EPAD, 4)
    g1m, g2m = _tc1(geo, w4a, w4b, b1g, b2g, pm)
    gg1 = _sc_gmean(g1m, kflat).reshape(NPAD, 64)
    P, x1, h2 = _tc2(g2m, gg1, w2at, w2bt, b2r, c1r, rbr, w1t, b1r)
    hm2 = _sc_gmean(h2, kflat).reshape(NPAD, 64)
    hh = _sc_gmean(hm2, kflat).reshape(NPAD, 64)
    x2 = _tc3(P, x1, hh, w2ct, wrt, brr)
    return x2[:N]
